# Initial kernel scaffold; baseline (speedup 1.0000x reference)
import jax
import jax.numpy as jnp
from jax.experimental import pallas as pl


def kernel(x, F0_rows, F0_cols, F0_vals, F1_rows, F1_cols, F1_vals, F2_rows, F2_cols, F2_vals, W, b):
    raise NotImplementedError("write your pallas kernel here")


# speedup vs baseline: 1.8969x; 1.8969x over previous
import functools

import jax
import jax.numpy as jnp
from jax import lax
from jax.experimental import pallas as pl
from jax.experimental.pallas import tpu as pltpu
from jax.experimental.pallas import tpu_sc as plsc

N = 10000
D = 256
H = 128
E3 = 480000
SUBC = 16
EDGES_PER_TILE = E3 // SUBC
CHUNK = 120
NCHUNKS = EDGES_PER_TILE // CHUNK
ROWS_PER_TILE = 624
ROWS_REM = N - SUBC * ROWS_PER_TILE


def _sc_spmm(x_lo, x_hi, cols, rows, vals, zrows):
    mesh = plsc.VectorSubcoreMesh(core_axis_name="c", subcore_axis_name="s")

    @functools.partial(
        pl.kernel,
        mesh=mesh,
        out_type=(
            jax.ShapeDtypeStruct((N, H), jnp.float32),
            jax.ShapeDtypeStruct((N, H), jnp.float32),
        ),
        scratch_types=[
            pltpu.VMEM((CHUNK,), jnp.int32),
            pltpu.VMEM((CHUNK,), jnp.int32),
            pltpu.VMEM((CHUNK, 16), jnp.float32),
            pltpu.VMEM((CHUNK, H), jnp.float32),
            pltpu.VMEM_SHARED((N, H), jnp.float32),
            pltpu.SemaphoreType.DMA,
        ],
    )
    def k(xlo_hbm, xhi_hbm, cols_hbm, rows_hbm, vals_hbm, zrows_hbm,
          alo_hbm, ahi_hbm, cols_v, rows_v, vals_v, g_v, acc, sem):
        c = lax.axis_index("c")
        s = lax.axis_index("s")
        r0 = s * ROWS_PER_TILE

        pltpu.sync_copy(zrows_hbm, acc.at[pl.ds(r0, ROWS_PER_TILE)])

        @pl.when(s == 0)
        def _():
            pltpu.sync_copy(zrows_hbm.at[pl.ds(0, ROWS_REM)],
                            acc.at[pl.ds(SUBC * ROWS_PER_TILE, ROWS_REM)])

        plsc.subcore_barrier()

        def edge_pass(x_hbm):
            def chunk_body(j, carry):
                off = s * EDGES_PER_TILE + j * CHUNK
                pltpu.sync_copy(cols_hbm.at[pl.ds(off, CHUNK)], cols_v)
                pltpu.sync_copy(rows_hbm.at[pl.ds(off, CHUNK)], rows_v)
                pltpu.sync_copy(vals_hbm.at[pl.ds(off, CHUNK)], vals_v)
                pltpu.async_copy(x_hbm.at[cols_v], g_v, sem).wait()

                def scale_row(i, carry2):
                    vv = vals_v[i, :]
                    for r in range(H // 16):
                        sl = pl.ds(r * 16, 16)
                        g_v[i, sl] = g_v[i, sl] * vv
                    return carry2

                lax.fori_loop(0, CHUNK, scale_row, 0)
                pltpu.sync_copy(g_v, acc.at[rows_v], add=True)
                return carry

            lax.fori_loop(0, NCHUNKS, chunk_body, 0)

        @pl.when(c == 0)
        def _():
            edge_pass(xlo_hbm)

        @pl.when(c == 1)
        def _():
            edge_pass(xhi_hbm)

        plsc.subcore_barrier()

        def wb(a_hbm):
            pltpu.sync_copy(acc.at[pl.ds(r0, ROWS_PER_TILE)],
                            a_hbm.at[pl.ds(r0, ROWS_PER_TILE)])

            @pl.when(s == 0)
            def _():
                tail = SUBC * ROWS_PER_TILE
                pltpu.sync_copy(acc.at[pl.ds(tail, ROWS_REM)],
                                a_hbm.at[pl.ds(tail, ROWS_REM)])

        @pl.when(c == 0)
        def _():
            wb(alo_hbm)

        @pl.when(c == 1)
        def _():
            wb(ahi_hbm)

    return k(x_lo, x_hi, cols, rows, vals, zrows)


def _tc_linear(a_lo, a_hi, w_lo, w_hi, b2):
    BM = 1000

    def mm(alo_ref, ahi_ref, wlo_ref, whi_ref, b_ref, o_ref):
        acc = jnp.dot(alo_ref[...], wlo_ref[...],
                      preferred_element_type=jnp.float32)
        acc = acc + jnp.dot(ahi_ref[...], whi_ref[...],
                            preferred_element_type=jnp.float32)
        o_ref[...] = acc + b_ref[...]

    return pl.pallas_call(
        mm,
        grid=(N // BM,),
        in_specs=[
            pl.BlockSpec((BM, H), lambda m: (m, 0)),
            pl.BlockSpec((BM, H), lambda m: (m, 0)),
            pl.BlockSpec((H, D), lambda m: (0, 0)),
            pl.BlockSpec((H, D), lambda m: (0, 0)),
            pl.BlockSpec((1, D), lambda m: (0, 0)),
        ],
        out_specs=pl.BlockSpec((BM, D), lambda m: (m, 0)),
        out_shape=jax.ShapeDtypeStruct((N, D), jnp.float32),
    )(a_lo, a_hi, w_lo, w_hi, b2)


@jax.jit
def kernel(x, F0_rows, F0_cols, F0_vals, F1_rows, F1_cols, F1_vals,
           F2_rows, F2_cols, F2_vals, W, b):
    cols = jnp.concatenate([F0_cols, F1_cols, F2_cols])
    rows = jnp.concatenate([F0_rows, F1_rows, F2_rows])
    vals = jnp.concatenate([F0_vals, F1_vals, F2_vals])
    vals = jnp.broadcast_to(vals[:, None], (E3, 16))
    x_lo = x[:, :H]
    x_hi = x[:, H:]
    zrows = jnp.zeros((ROWS_PER_TILE, H), dtype=jnp.float32)
    a_lo, a_hi = _sc_spmm(x_lo, x_hi, cols, rows, vals, zrows)
    return _tc_linear(a_lo, a_hi, W[:H], W[H:], b.reshape(1, D))


# speedup vs baseline: 5.4724x; 2.8849x over previous
import functools

import jax
import jax.numpy as jnp
from jax import lax
from jax.experimental import pallas as pl
from jax.experimental.pallas import tpu as pltpu
from jax.experimental.pallas import tpu_sc as plsc

N = 10000
D = 256
H = 128
E3 = 480000
SUBC = 16
EDGES_PER_TILE = E3 // SUBC
CHUNK = 80
NCHUNKS = EDGES_PER_TILE // CHUNK
NCH_ALL = E3 // CHUNK
NSLOT = 3
NPAIR = NCHUNKS // NSLOT
ROWS_PER_TILE = 624
ROWS_REM = N - SUBC * ROWS_PER_TILE
UNROLL = 8
VROWS = CHUNK * 16 // 128


def _sc_spmm(x_lo, x_hi, cols, rows, vals3, zrows):
    mesh = plsc.VectorSubcoreMesh(core_axis_name="c", subcore_axis_name="s")

    @functools.partial(
        pl.kernel,
        mesh=mesh,
        out_type=(
            jax.ShapeDtypeStruct((N, H), jnp.float32),
            jax.ShapeDtypeStruct((N, H), jnp.float32),
        ),
        scratch_types=[
            pltpu.VMEM((NSLOT, CHUNK), jnp.int32),
            pltpu.VMEM((NSLOT, CHUNK), jnp.int32),
            pltpu.VMEM((NSLOT, CHUNK), jnp.int32),
            pltpu.VMEM((NSLOT, VROWS, 128), jnp.float32),
            pltpu.VMEM((NSLOT, CHUNK, H), jnp.float32),
            pltpu.VMEM_SHARED((N, H), jnp.float32),
            [pltpu.SemaphoreType.DMA] * NSLOT,
            [pltpu.SemaphoreType.DMA] * NSLOT,
            [pltpu.SemaphoreType.DMA] * NSLOT,
        ],
    )
    def k(xlo_hbm, xhi_hbm, cols_hbm, rows_hbm, vals_hbm, zrows_hbm,
          alo_hbm, ahi_hbm, colsb, rowsb, rows_s, vals_v, g_v, acc,
          sg, sv, ss):
        c = lax.axis_index("c")
        s = lax.axis_index("s")
        r0 = s * ROWS_PER_TILE
        e0 = s * EDGES_PER_TILE
        c0 = s * NCHUNKS

        pltpu.sync_copy(zrows_hbm, acc.at[pl.ds(r0, ROWS_PER_TILE)])

        @pl.when(s == 0)
        def _():
            pltpu.sync_copy(zrows_hbm.at[pl.ds(0, ROWS_REM)],
                            acc.at[pl.ds(SUBC * ROWS_PER_TILE, ROWS_REM)])

        plsc.subcore_barrier()

        def edge_pass(x_hbm):
            def start_meta(j, p):
                off = e0 + j * CHUNK
                pltpu.async_copy(cols_hbm.at[pl.ds(off, CHUNK)],
                                 colsb.at[p], sv[p])
                pltpu.async_copy(rows_hbm.at[pl.ds(off, CHUNK)],
                                 rowsb.at[p], sv[p])
                pltpu.async_copy(vals_hbm.at[c0 + j], vals_v.at[p], sv[p])

            def wait_meta(p):
                pltpu.make_async_copy(cols_hbm.at[pl.ds(e0, CHUNK)],
                                      colsb.at[p], sv[p]).wait()
                pltpu.make_async_copy(rows_hbm.at[pl.ds(e0, CHUNK)],
                                      rowsb.at[p], sv[p]).wait()
                pltpu.make_async_copy(vals_hbm.at[c0], vals_v.at[p],
                                      sv[p]).wait()

            def start_gather(p):
                pltpu.async_copy(x_hbm.at[colsb.at[p]], g_v.at[p], sg[p])

            def wait_gather(p):
                pltpu.make_async_copy(x_hbm.at[colsb.at[p]], g_v.at[p],
                                      sg[p]).wait()

            def start_scat(p):
                pltpu.async_copy(g_v.at[p], acc.at[rows_s.at[p]], ss[p],
                                 add=True)

            def wait_scat(p):
                pltpu.make_async_copy(g_v.at[p], acc.at[rows_s.at[p]],
                                      ss[p]).wait()

            def scale(p):
                def scale_blk(ii, carry2):
                    for u in range(UNROLL):
                        i = ii * UNROLL + u
                        vv = vals_v[p, ii, pl.ds(u * 16, 16)]
                        for r in range(H // 16):
                            sl = pl.ds(r * 16, 16)
                            g_v[p, i, sl] = g_v[p, i, sl] * vv
                    return carry2

                lax.fori_loop(0, CHUNK // UNROLL, scale_blk, 0,
                              unroll=False)

            for p in range(NSLOT):
                start_meta(p, p)
            wait_meta(0)
            start_gather(0)

            def iter_body(t, carry):
                for u in range(NSLOT):
                    j = NSLOT * t + u
                    v = (u + 1) % NSLOT

                    def prefetch():
                        if u == NSLOT - 1:
                            wait_scat(v)
                        else:
                            @pl.when(t >= 1)
                            def _():
                                wait_scat(v)

                        wait_meta(v)
                        start_gather(v)

                    if u == NSLOT - 1:
                        @pl.when(t < NPAIR - 1)
                        def _():
                            prefetch()
                    else:
                        prefetch()

                    wait_gather(u)
                    scale(u)
                    for r in range(CHUNK // 16):
                        sl = pl.ds(r * 16, 16)
                        rows_s[u, sl] = rowsb[u, sl]
                    start_scat(u)

                    @pl.when(t < NPAIR - 1)
                    def _():
                        start_meta(j + NSLOT, u)
                return carry

            lax.fori_loop(0, NPAIR, iter_body, 0)
            for p in range(NSLOT):
                wait_scat(p)

        @pl.when(c == 0)
        def _():
            edge_pass(xlo_hbm)

        @pl.when(c == 1)
        def _():
            edge_pass(xhi_hbm)

        plsc.subcore_barrier()

        def wb(a_hbm):
            pltpu.sync_copy(acc.at[pl.ds(r0, ROWS_PER_TILE)],
                            a_hbm.at[pl.ds(r0, ROWS_PER_TILE)])

            @pl.when(s == 0)
            def _():
                tail = SUBC * ROWS_PER_TILE
                pltpu.sync_copy(acc.at[pl.ds(tail, ROWS_REM)],
                                a_hbm.at[pl.ds(tail, ROWS_REM)])

        @pl.when(c == 0)
        def _():
            wb(alo_hbm)

        @pl.when(c == 1)
        def _():
            wb(ahi_hbm)

    return k(x_lo, x_hi, cols, rows, vals3, zrows)


def _tc_linear(a_lo, a_hi, w_lo, w_hi, b2):
    BM = 1000

    def mm(alo_ref, ahi_ref, wlo_ref, whi_ref, b_ref, o_ref):
        acc = jnp.dot(alo_ref[...], wlo_ref[...],
                      preferred_element_type=jnp.float32)
        acc = acc + jnp.dot(ahi_ref[...], whi_ref[...],
                            preferred_element_type=jnp.float32)
        o_ref[...] = acc + b_ref[...]

    return pl.pallas_call(
        mm,
        grid=(N // BM,),
        in_specs=[
            pl.BlockSpec((BM, H), lambda m: (m, 0)),
            pl.BlockSpec((BM, H), lambda m: (m, 0)),
            pl.BlockSpec((H, D), lambda m: (0, 0)),
            pl.BlockSpec((H, D), lambda m: (0, 0)),
            pl.BlockSpec((1, D), lambda m: (0, 0)),
        ],
        out_specs=pl.BlockSpec((BM, D), lambda m: (m, 0)),
        out_shape=jax.ShapeDtypeStruct((N, D), jnp.float32),
    )(a_lo, a_hi, w_lo, w_hi, b2)


@jax.jit
def kernel(x, F0_rows, F0_cols, F0_vals, F1_rows, F1_cols, F1_vals,
           F2_rows, F2_cols, F2_vals, W, b):
    cols = jnp.concatenate([F0_cols, F1_cols, F2_cols])
    rows = jnp.concatenate([F0_rows, F1_rows, F2_rows])
    vals = jnp.concatenate([F0_vals, F1_vals, F2_vals])
    vals3 = jnp.broadcast_to(
        vals.reshape(NCH_ALL, CHUNK, 1),
        (NCH_ALL, CHUNK, 16)).reshape(NCH_ALL, VROWS, 128)
    x_lo = x[:, :H]
    x_hi = x[:, H:]
    zrows = jnp.zeros((ROWS_PER_TILE, H), dtype=jnp.float32)
    a_lo, a_hi = _sc_spmm(x_lo, x_hi, cols, rows, vals3, zrows)
    return _tc_linear(a_lo, a_hi, W[:H], W[H:], b.reshape(1, D))
